# baseline (speedup 1.0000x reference)
import jax
import jax.numpy as jnp
from jax import lax
from jax.experimental import pallas as pl

D = 256
K = 64
BN = 256


def _prep_kernel(xkT_ref, dpT_ref, sc_ref, sh_ref, xkt_ref, wt_ref):
    f32 = jnp.float32
    xkT = xkT_ref[...]
    dpT = dpT_ref[...]
    row = lax.broadcasted_iota(jnp.int32, (K, K), 0)
    col = lax.broadcasted_iota(jnp.int32, (K, K), 1)
    rmask = lax.broadcasted_iota(jnp.int32, (K, D), 0)

    m1 = (col == row + 1).astype(f32) - (col == row).astype(f32)
    dxT = jnp.dot(m1, xkT, preferred_element_type=f32)
    dxT = jnp.where(rmask == K - 1, 0.0, dxT)

    sT = jax.nn.softplus(dpT) + 1e-4
    avg = jnp.sum(sT * dxT, axis=0, keepdims=True) / (
        jnp.sum(dxT, axis=0, keepdims=True) + 1e-8)
    sT = sT / (avg + 1e-8)

    scale = jax.nn.softplus(sc_ref[...]) + 1e-3
    shiftv = sh_ref[...]

    d2 = (col == row).astype(f32) - (col == row - 1).astype(f32)
    ds = jnp.dot(d2, sT, preferred_element_type=f32)
    w = -scale * ds

    s0 = jnp.sum(jnp.where(rmask == 0, sT, 0.0), axis=0, keepdims=True)
    s_last = jnp.sum(jnp.where(rmask == K - 2, sT, 0.0), axis=0, keepdims=True)
    xk0 = jnp.sum(jnp.where(rmask == 0, xkT, 0.0), axis=0, keepdims=True)
    c1 = scale * s_last
    c0 = shiftv - scale * s0 * xk0

    wt = jnp.where(rmask == 0, c1, jnp.where(rmask == K - 1, c0, w))
    wt_ref[...] = wt
    xkt_ref[...] = xkT


def _scan_kernel(x_ref, xkt_ref, wt_ref, o_ref):
    x = x_ref[...]
    xkt = xkt_ref[...]
    wt = wt_ref[...]
    acc = x * wt[0, :][None, :] + wt[K - 1, :][None, :]
    for j in range(1, K - 1):
        acc = acc + wt[j, :][None, :] * jnp.minimum(x, xkt[j, :][None, :])
    o_ref[...] = acc


def kernel(x, xk, delta_raw, scale_raw, shift):
    f32 = jnp.float32
    n = x.shape[0]
    xkT = xk.T.astype(f32)
    dpT = jnp.pad(delta_raw, ((0, 0), (0, 1))).T.astype(f32)
    sc = scale_raw[None, :].astype(f32)
    sh = shift[None, :].astype(f32)

    xkt, wt = pl.pallas_call(
        _prep_kernel,
        out_shape=[jax.ShapeDtypeStruct((K, D), f32)] * 2,
    )(xkT, dpT, sc, sh)

    tab_spec = pl.BlockSpec((K, D), lambda i: (0, 0))
    out = pl.pallas_call(
        _scan_kernel,
        grid=(n // BN,),
        in_specs=[pl.BlockSpec((BN, D), lambda i: (i, 0)),
                  tab_spec, tab_spec],
        out_specs=pl.BlockSpec((BN, D), lambda i: (i, 0)),
        out_shape=jax.ShapeDtypeStruct((n, D), f32),
    )(x, xkt, wt)
    return out


# speedup vs baseline: 1.0139x; 1.0139x over previous
import jax
import jax.numpy as jnp
from jax import lax
from jax.experimental import pallas as pl

D = 256
K = 64
BN = 1024


def _prep_kernel(xkT_ref, dpT_ref, sc_ref, sh_ref, xkt_ref, wt_ref):
    f32 = jnp.float32
    xkT = xkT_ref[...]
    dpT = dpT_ref[...]
    row = lax.broadcasted_iota(jnp.int32, (K, K), 0)
    col = lax.broadcasted_iota(jnp.int32, (K, K), 1)
    rmask = lax.broadcasted_iota(jnp.int32, (K, D), 0)

    m1 = (col == row + 1).astype(f32) - (col == row).astype(f32)
    dxT = jnp.dot(m1, xkT, preferred_element_type=f32)
    dxT = jnp.where(rmask == K - 1, 0.0, dxT)

    sT = jax.nn.softplus(dpT) + 1e-4
    avg = jnp.sum(sT * dxT, axis=0, keepdims=True) / (
        jnp.sum(dxT, axis=0, keepdims=True) + 1e-8)
    sT = sT / (avg + 1e-8)

    scale = jax.nn.softplus(sc_ref[...]) + 1e-3
    shiftv = sh_ref[...]

    d2 = (col == row).astype(f32) - (col == row - 1).astype(f32)
    ds = jnp.dot(d2, sT, preferred_element_type=f32)
    w = -scale * ds

    s0 = jnp.sum(jnp.where(rmask == 0, sT, 0.0), axis=0, keepdims=True)
    s_last = jnp.sum(jnp.where(rmask == K - 2, sT, 0.0), axis=0, keepdims=True)
    xk0 = jnp.sum(jnp.where(rmask == 0, xkT, 0.0), axis=0, keepdims=True)
    c1 = scale * s_last
    c0 = shiftv - scale * s0 * xk0

    wt = jnp.where(rmask == 0, c1, jnp.where(rmask == K - 1, c0, w))
    wt_ref[...] = wt
    xkt_ref[...] = xkT


def _scan_kernel(x_ref, xkt_ref, wt_ref, o_ref):
    x = x_ref[...]
    xkt = xkt_ref[...]
    wt = wt_ref[...]
    acc = x * wt[0, :][None, :] + wt[K - 1, :][None, :]
    for j in range(1, K - 1):
        acc = acc + wt[j, :][None, :] * jnp.minimum(x, xkt[j, :][None, :])
    o_ref[...] = acc


def kernel(x, xk, delta_raw, scale_raw, shift):
    f32 = jnp.float32
    n = x.shape[0]
    xkT = xk.T.astype(f32)
    dpT = jnp.pad(delta_raw, ((0, 0), (0, 1))).T.astype(f32)
    sc = scale_raw[None, :].astype(f32)
    sh = shift[None, :].astype(f32)

    xkt, wt = pl.pallas_call(
        _prep_kernel,
        out_shape=[jax.ShapeDtypeStruct((K, D), f32)] * 2,
    )(xkT, dpT, sc, sh)

    tab_spec = pl.BlockSpec((K, D), lambda i: (0, 0))
    out = pl.pallas_call(
        _scan_kernel,
        grid=(n // BN,),
        in_specs=[pl.BlockSpec((BN, D), lambda i: (i, 0)),
                  tab_spec, tab_spec],
        out_specs=pl.BlockSpec((BN, D), lambda i: (i, 0)),
        out_shape=jax.ShapeDtypeStruct((n, D), f32),
    )(x, xkt, wt)
    return out
